# Initial kernel scaffold; baseline (speedup 1.0000x reference)
import jax
import jax.numpy as jnp
from jax.experimental import pallas as pl


def kernel(year, answer, answer_table, yearly_table, question_table, alpha, beta):
    raise NotImplementedError("write your pallas kernel here")


# speedup vs baseline: 3.3486x; 3.3486x over previous
import functools

import jax
import jax.numpy as jnp
from jax import lax
from jax.experimental import pallas as pl
from jax.experimental.pallas import tpu as pltpu
from jax.experimental.pallas import tpu_sc as plsc

VOCAB = 1000
NQ = 100
NY = 14
D = 128
B = 4096

NC = 2
NS = 16
L = 16
NW = NC * NS
BW = B // NW
ND = D // L


def _body(year_h, ans_h, atab_h, ytab_h, qtab_h, a16_h, b16_h, out_h,
          idx_v, yidx_v, yrow_v, q_v, rows_v, a_v, b_v, gsem):
  wid = lax.axis_index("s") * NC + lax.axis_index("c")
  base = wid * BW

  pltpu.sync_copy(ans_h.at[pl.ds(base, BW)], idx_v)
  pltpu.sync_copy(year_h.at[pl.ds(base, BW)], yidx_v)
  pltpu.sync_copy(qtab_h, q_v)
  pltpu.sync_copy(a16_h, a_v)
  pltpu.sync_copy(b16_h, b_v)
  pltpu.async_copy(ytab_h.at[yidx_v], yrow_v, gsem).wait()

  alpha = a_v[...]
  beta = b_v[...]

  def scale_yr(i, carry):
    for d in range(ND):
      yrow_v[i, pl.ds(d * L, L)] = yrow_v[i, pl.ds(d * L, L)] * alpha
    return carry
  lax.fori_loop(0, BW, scale_yr, 0, unroll=False)

  def scale_q(i, carry):
    for d in range(ND):
      q_v[i, pl.ds(d * L, L)] = q_v[i, pl.ds(d * L, L)] * beta
    return carry
  lax.fori_loop(0, NQ, scale_q, 0, unroll=False)

  def body(i, carry):
    pltpu.async_copy(atab_h.at[idx_v.at[i]], rows_v, gsem).wait()
    yr = [yrow_v[i, pl.ds(d * L, L)] for d in range(ND)]

    def qloop(q, c):
      for d in range(ND):
        t = q_v[q, pl.ds(d * L, L)] + yr[d]
        plsc.addupdate(rows_v.at[q, pl.ds(d * L, L)], t)
      return c
    lax.fori_loop(0, NQ, qloop, 0, unroll=False)

    pltpu.sync_copy(rows_v, out_h.at[base + i])
    return carry
  lax.fori_loop(0, BW, body, 0, unroll=False)


@jax.jit
def _sc_call(year, answer, answer_table, yearly_table, question_table,
             a16, b16):
  mesh = plsc.VectorSubcoreMesh(
      core_axis_name="c", subcore_axis_name="s",
      num_cores=NC, num_subcores=NS)
  f = pl.kernel(
      _body, mesh=mesh,
      out_type=jax.ShapeDtypeStruct((B, NQ, D), jnp.float32),
      scratch_types=[
          pltpu.VMEM((BW, NQ), jnp.int32),
          pltpu.VMEM((BW,), jnp.int32),
          pltpu.VMEM((BW, D), jnp.float32),
          pltpu.VMEM((NQ, D), jnp.float32),
          pltpu.VMEM((NQ, D), jnp.float32),
          pltpu.VMEM((L,), jnp.float32),
          pltpu.VMEM((L,), jnp.float32),
          pltpu.SemaphoreType.DMA,
      ])
  return f(year, answer, answer_table, yearly_table, question_table,
           a16, b16)


def kernel(year, answer, answer_table, yearly_table, question_table,
           alpha, beta):
  a16 = jnp.broadcast_to(alpha.astype(jnp.float32), (L,))
  b16 = jnp.broadcast_to(beta.astype(jnp.float32), (L,))
  return _sc_call(year.astype(jnp.int32), answer.astype(jnp.int32),
                  answer_table, yearly_table, question_table, a16, b16)


# speedup vs baseline: 4.5428x; 1.3566x over previous
import jax
import jax.numpy as jnp
from jax import lax
from jax.experimental import pallas as pl
from jax.experimental.pallas import tpu as pltpu
from jax.experimental.pallas import tpu_sc as plsc

VOCAB = 1000
NQ = 100
NY = 14
D = 128
B = 4096

NC = 2
NS = 16
L = 16
NW = NC * NS
BW = B // NW
ND = D // L
NSLOT = 4


def _body(year_h, ans_h, atab_h, ytab_h, qtab_h, a16_h, b16_h, out_h,
          idx_v, yidx_v, yrow_v, q_v, rows_v, a_v, b_v, gsems, ssems):
  wid = lax.axis_index("s") * NC + lax.axis_index("c")
  base = wid * BW

  pltpu.sync_copy(ans_h.at[pl.ds(base, BW)], idx_v)
  pltpu.sync_copy(year_h.at[pl.ds(base, BW)], yidx_v)
  pltpu.sync_copy(qtab_h, q_v)
  pltpu.sync_copy(a16_h, a_v)
  pltpu.sync_copy(b16_h, b_v)
  pltpu.async_copy(ytab_h.at[yidx_v], yrow_v, gsems[0]).wait()

  alpha = a_v[...]
  beta = b_v[...]

  def scale_yr(i, carry):
    for d in range(ND):
      yrow_v[i, pl.ds(d * L, L)] = yrow_v[i, pl.ds(d * L, L)] * alpha
    return carry
  lax.fori_loop(0, BW, scale_yr, 0, unroll=False)

  def scale_q(i, carry):
    for d in range(ND):
      q_v[i, pl.ds(d * L, L)] = q_v[i, pl.ds(d * L, L)] * beta
    return carry
  lax.fori_loop(0, NQ, scale_q, 0, unroll=False)

  def gather_start(i, s):
    pltpu.make_async_copy(atab_h.at[idx_v.at[i]], rows_v.at[s],
                          gsems[s]).start()

  def gather_wait(i, s):
    pltpu.make_async_copy(atab_h.at[idx_v.at[i]], rows_v.at[s],
                          gsems[s]).wait()

  def scatter_start(i, s):
    pltpu.make_async_copy(rows_v.at[s], out_h.at[base + i], ssems[s]).start()

  def scatter_wait(s):
    pltpu.make_async_copy(rows_v.at[s], out_h.at[base], ssems[s]).wait()

  def compute(i, s):
    yr = [yrow_v[i, pl.ds(d * L, L)] for d in range(ND)]

    def qloop(q, c):
      for d in range(ND):
        t = q_v[q, pl.ds(d * L, L)] + yr[d]
        plsc.addupdate(rows_v.at[s, q, pl.ds(d * L, L)], t)
      return c
    lax.fori_loop(0, NQ, qloop, 0, unroll=4)

  def process(i, s):
    gather_wait(i, s)
    compute(i, s)
    scatter_start(i, s)

  for s in range(NSLOT - 1):
    gather_start(s, s)
  process(0, 0)
  gather_start(3, 3)
  for i in range(1, NSLOT):
    process(i, i)
    sp = (i + 3) % NSLOT
    scatter_wait(sp)
    gather_start(i + 3, sp)

  def outer(io, c):
    for s in range(NSLOT):
      i = io * NSLOT + s
      process(i, s)
      j = i + 3
      sp = (s + 3) % NSLOT

      @pl.when(j < BW)
      def _():
        scatter_wait(sp)
        gather_start(j, sp)
    return c
  lax.fori_loop(1, BW // NSLOT, outer, 0, unroll=False)

  for s in range(NSLOT):
    scatter_wait(s)


@jax.jit
def _sc_call(year, answer, answer_table, yearly_table, question_table,
             a16, b16):
  mesh = plsc.VectorSubcoreMesh(
      core_axis_name="c", subcore_axis_name="s",
      num_cores=NC, num_subcores=NS)
  f = pl.kernel(
      _body, mesh=mesh,
      out_type=jax.ShapeDtypeStruct((B, NQ, D), jnp.float32),
      scratch_types=[
          pltpu.VMEM((BW, NQ), jnp.int32),
          pltpu.VMEM((BW,), jnp.int32),
          pltpu.VMEM((BW, D), jnp.float32),
          pltpu.VMEM((NQ, D), jnp.float32),
          pltpu.VMEM((NSLOT, NQ, D), jnp.float32),
          pltpu.VMEM((L,), jnp.float32),
          pltpu.VMEM((L,), jnp.float32),
          [pltpu.SemaphoreType.DMA] * NSLOT,
          [pltpu.SemaphoreType.DMA] * NSLOT,
      ])
  return f(year, answer, answer_table, yearly_table, question_table,
           a16, b16)


def kernel(year, answer, answer_table, yearly_table, question_table,
           alpha, beta):
  a16 = jnp.broadcast_to(alpha.astype(jnp.float32), (L,))
  b16 = jnp.broadcast_to(beta.astype(jnp.float32), (L,))
  return _sc_call(year.astype(jnp.int32), answer.astype(jnp.int32),
                  answer_table, yearly_table, question_table, a16, b16)


# speedup vs baseline: 6.1999x; 1.3648x over previous
import jax
import jax.numpy as jnp
from jax import lax
from jax.experimental import pallas as pl
from jax.experimental.pallas import tpu as pltpu
from jax.experimental.pallas import tpu_sc as plsc

VOCAB = 1000
NQ = 100
NY = 14
D = 128
B = 4096

NC = 2
NS = 16
L = 16
NW = NC * NS
BW = B // NW
ND = D // L
NSLOT = 4


def _body(year_h, ans_h, atab_h, ytab_h, qtab_h, a16_h, b16_h, out_h,
          idx_v, yidx_v, yrow_v, q_v, rows_v, a_v, b_v, atab_s,
          gsems, ssems):
  sid = lax.axis_index("s")
  wid = sid * NC + lax.axis_index("c")
  base = wid * BW

  rows_per_tile = 64
  tab_lo = pl.multiple_of(sid * rows_per_tile, 8)
  rem = VOCAB - (NS - 1) * rows_per_tile

  @pl.when(sid < NS - 1)
  def _():
    pltpu.sync_copy(atab_h.at[pl.ds(tab_lo, rows_per_tile)],
                    atab_s.at[pl.ds(tab_lo, rows_per_tile)])

  @pl.when(sid == NS - 1)
  def _():
    pltpu.sync_copy(atab_h.at[pl.ds((NS - 1) * rows_per_tile, rem)],
                    atab_s.at[pl.ds((NS - 1) * rows_per_tile, rem)])

  pltpu.sync_copy(ans_h.at[pl.ds(base, BW)], idx_v)
  pltpu.sync_copy(year_h.at[pl.ds(base, BW)], yidx_v)
  pltpu.sync_copy(qtab_h, q_v)
  pltpu.sync_copy(a16_h, a_v)
  pltpu.sync_copy(b16_h, b_v)
  pltpu.async_copy(ytab_h.at[yidx_v], yrow_v, gsems[0]).wait()

  plsc.subcore_barrier()

  alpha = a_v[...]
  beta = b_v[...]

  def scale_yr(i, carry):
    for d in range(ND):
      yrow_v[i, pl.ds(d * L, L)] = yrow_v[i, pl.ds(d * L, L)] * alpha
    return carry
  lax.fori_loop(0, BW, scale_yr, 0, unroll=False)

  def scale_q(i, carry):
    for d in range(ND):
      q_v[i, pl.ds(d * L, L)] = q_v[i, pl.ds(d * L, L)] * beta
    return carry
  lax.fori_loop(0, NQ, scale_q, 0, unroll=False)

  def gather_start(i, s):
    pltpu.make_async_copy(atab_s.at[idx_v.at[i]], rows_v.at[s],
                          gsems[s]).start()

  def gather_wait(i, s):
    pltpu.make_async_copy(atab_s.at[idx_v.at[i]], rows_v.at[s],
                          gsems[s]).wait()

  def scatter_start(i, s):
    pltpu.make_async_copy(rows_v.at[s], out_h.at[base + i], ssems[s]).start()

  def scatter_wait(s):
    pltpu.make_async_copy(rows_v.at[s], out_h.at[base], ssems[s]).wait()

  def compute(i, s):
    yr = [yrow_v[i, pl.ds(d * L, L)] for d in range(ND)]

    def qloop(q, c):
      for d in range(ND):
        t = q_v[q, pl.ds(d * L, L)] + yr[d]
        plsc.addupdate(rows_v.at[s, q, pl.ds(d * L, L)], t)
      return c
    lax.fori_loop(0, NQ, qloop, 0, unroll=4)

  def process(i, s):
    gather_wait(i, s)
    compute(i, s)
    scatter_start(i, s)

  for s in range(NSLOT - 1):
    gather_start(s, s)
  process(0, 0)
  gather_start(3, 3)
  for i in range(1, NSLOT):
    process(i, i)
    sp = (i + 3) % NSLOT
    scatter_wait(sp)
    gather_start(i + 3, sp)

  def outer(io, c):
    for s in range(NSLOT):
      i = io * NSLOT + s
      process(i, s)
      j = i + 3
      sp = (s + 3) % NSLOT

      @pl.when(j < BW)
      def _():
        scatter_wait(sp)
        gather_start(j, sp)
    return c
  lax.fori_loop(1, BW // NSLOT, outer, 0, unroll=False)

  for s in range(NSLOT):
    scatter_wait(s)


@jax.jit
def _sc_call(year, answer, answer_table, yearly_table, question_table,
             a16, b16):
  mesh = plsc.VectorSubcoreMesh(
      core_axis_name="c", subcore_axis_name="s",
      num_cores=NC, num_subcores=NS)
  f = pl.kernel(
      _body, mesh=mesh,
      out_type=jax.ShapeDtypeStruct((B, NQ, D), jnp.float32),
      scratch_types=[
          pltpu.VMEM((BW, NQ), jnp.int32),
          pltpu.VMEM((BW,), jnp.int32),
          pltpu.VMEM((BW, D), jnp.float32),
          pltpu.VMEM((NQ, D), jnp.float32),
          pltpu.VMEM((NSLOT, NQ, D), jnp.float32),
          pltpu.VMEM((L,), jnp.float32),
          pltpu.VMEM((L,), jnp.float32),
          pltpu.VMEM_SHARED((VOCAB, D), jnp.float32),
          [pltpu.SemaphoreType.DMA] * NSLOT,
          [pltpu.SemaphoreType.DMA] * NSLOT,
      ])
  return f(year, answer, answer_table, yearly_table, question_table,
           a16, b16)


def kernel(year, answer, answer_table, yearly_table, question_table,
           alpha, beta):
  a16 = jnp.broadcast_to(alpha.astype(jnp.float32), (L,))
  b16 = jnp.broadcast_to(beta.astype(jnp.float32), (L,))
  return _sc_call(year.astype(jnp.int32), answer.astype(jnp.int32),
                  answer_table, yearly_table, question_table, a16, b16)
